# Initial kernel scaffold; baseline (speedup 1.0000x reference)
import jax
import jax.numpy as jnp
from jax.experimental import pallas as pl


def kernel(p, x, o, W, gamma, beta):
    raise NotImplementedError("write your pallas kernel here")


# speedup vs baseline: 28.0491x; 28.0491x over previous
import functools

import jax
import jax.numpy as jnp
from jax import lax
from jax.experimental import pallas as pl
from jax.experimental.pallas import tpu as pltpu
from jax.experimental.pallas import tpu_sc as plsc

N = 16384
N_NEW = 4096
K = 16
OUT = 256
FAN_IN = 131
EPS = 1e-5

SC_CORES = 2
SC_SUBCORES = 16
NW = SC_CORES * SC_SUBCORES

BIG_I32 = 2 ** 30


def _fps_body(px_ref, py_ref, pz_ref, out_ref):
    fi_out = lax.broadcasted_iota(jnp.int32, (32, 128), 0) * 128 + \
        lax.broadcasted_iota(jnp.int32, (32, 128), 1)
    fi_p = lax.broadcasted_iota(jnp.int32, (128, 128), 0) * 128 + \
        lax.broadcasted_iota(jnp.int32, (128, 128), 1)
    px = px_ref[...]
    py = py_ref[...]
    pz = pz_ref[...]

    out_ref[...] = jnp.where(fi_out == 0, jnp.int32(0), jnp.int32(-1))
    qx0 = jnp.sum(jnp.where(fi_p == 0, px, 0.0))
    qy0 = jnp.sum(jnp.where(fi_p == 0, py, 0.0))
    qz0 = jnp.sum(jnp.where(fi_p == 0, pz, 0.0))

    def cond(c):
        i, _, _, _, _, _, found = c
        return jnp.logical_and(i < N_NEW, jnp.logical_not(found))

    def body(c):
        i, qx, qy, qz, _, _, _ = c
        d = (px - qx) ** 2 + (py - qy) ** 2 + (pz - qz) ** 2
        m = jnp.max(d)
        nxt = jnp.min(jnp.where(d == m, fi_p, BIG_I32))
        chain = out_ref[...]
        occ = jnp.min(jnp.where(chain == nxt, fi_out, BIG_I32))
        found = occ < BIG_I32
        start = jnp.where(found, occ, 0)
        period = jnp.where(found, i - occ, 0)
        out_ref[...] = jnp.where(
            jnp.logical_and(fi_out == i, jnp.logical_not(found)), nxt, chain)
        nqx = jnp.sum(jnp.where(fi_p == nxt, px, 0.0))
        nqy = jnp.sum(jnp.where(fi_p == nxt, py, 0.0))
        nqz = jnp.sum(jnp.where(fi_p == nxt, pz, 0.0))
        i_next = jnp.where(found, i, i + 1)
        return (i_next, nqx, nqy, nqz, start, period, found)

    init = (jnp.int32(1), qx0, qy0, qz0, jnp.int32(0), jnp.int32(0),
            jnp.bool_(False))
    _, _, _, _, start, period, found = lax.while_loop(cond, body, init)

    def fill(j, _):
        chain = out_ref[...]
        val = jnp.sum(jnp.where(fi_out == start + j, chain, 0))
        mask = jnp.logical_and(fi_out >= start,
                               (fi_out - start) % period == j)
        out_ref[...] = jnp.where(mask, val, chain)
        return 0

    lax.fori_loop(0, period, fill, 0)


def _fps(px, py, pz):
    return pl.pallas_call(
        _fps_body,
        out_shape=jax.ShapeDtypeStruct((32, 128), jnp.int32),
    )(px, py, pz)


def _newp_gather(fps, p_pad):
    b_per = N_NEW // NW
    mesh = plsc.VectorSubcoreMesh(core_axis_name="c", subcore_axis_name="s")

    @functools.partial(
        pl.kernel,
        out_type=jax.ShapeDtypeStruct((N_NEW, 128), jnp.float32),
        mesh=mesh,
        scratch_types=[
            pltpu.VMEM((b_per,), jnp.int32),
            pltpu.VMEM((b_per, 128), jnp.float32),
            pltpu.SemaphoreType.DMA,
        ],
    )
    def body(fps_hbm, ppad_hbm, out_hbm, idx_v, rows_v, sem):
        wid = lax.axis_index("s") * SC_CORES + lax.axis_index("c")
        base = wid * b_per
        pltpu.sync_copy(fps_hbm.at[pl.ds(base, b_per)], idx_v)
        pltpu.async_copy(ppad_hbm.at[idx_v], rows_v, sem).wait()
        pltpu.sync_copy(rows_v, out_hbm.at[pl.ds(base, b_per)])

    return body(fps, p_pad)


_QT = 64


def _knn_body(np_ref, pt_ref, out_ref):
    pt = pt_ref[...]
    pn = jnp.sum(pt * pt, axis=0, keepdims=True)
    q = np_ref[...]
    qn = jnp.sum(q * q, axis=1, keepdims=True)
    dot = jax.lax.dot_general(q, pt, (((1,), (0,)), ((), ())),
                              preferred_element_type=jnp.float32)
    d = qn + pn - 2.0 * dot
    col = lax.broadcasted_iota(jnp.int32, (_QT, N), 1)
    inf = jnp.float32(jnp.inf)
    for r in range(K):
        m = jnp.min(d, axis=1, keepdims=True)
        sel = jnp.min(jnp.where(d == m, col, BIG_I32), axis=1, keepdims=True)
        out_ref[:, pl.ds(r, 1)] = sel
        d = jnp.where(col == sel, inf, d)


def _knn(np_pad, p_padT):
    grid = (N_NEW // _QT,)
    return pl.pallas_call(
        _knn_body,
        grid=grid,
        in_specs=[
            pl.BlockSpec((_QT, 16), lambda i: (i, 0)),
            pl.BlockSpec((16, N), lambda i: (0, 0)),
        ],
        out_specs=pl.BlockSpec((_QT, K), lambda i: (i, 0)),
        out_shape=jax.ShapeDtypeStruct((N_NEW, K), jnp.int32),
    )(np_pad, p_padT)


_RT = 2048


def _lin_body(xc_ref, w_ref, out_ref):
    out_ref[...] = jax.lax.dot_general(
        xc_ref[...], w_ref[...], (((1,), (1,)), ((), ())),
        preferred_element_type=jnp.float32)


def _linear(xcat, W):
    grid = (N // _RT,)
    return pl.pallas_call(
        _lin_body,
        grid=grid,
        in_specs=[
            pl.BlockSpec((_RT, FAN_IN), lambda i: (i, 0)),
            pl.BlockSpec((OUT, FAN_IN), lambda i: (0, 0)),
        ],
        out_specs=pl.BlockSpec((_RT, OUT), lambda i: (i, 0)),
        out_shape=jax.ShapeDtypeStruct((N, OUT), jnp.float32),
    )(xcat, W)


_G = 4
_RPG = _G * K


def _group_stage(idx_flat, T, zeros_i32):
    q_per = N_NEW // NW
    ng = q_per // _G
    mesh = plsc.VectorSubcoreMesh(core_axis_name="c", subcore_axis_name="s")

    @functools.partial(
        pl.kernel,
        out_type=(
            jax.ShapeDtypeStruct((N_NEW, OUT), jnp.float32),
            jax.ShapeDtypeStruct((N_NEW, OUT), jnp.float32),
            jax.ShapeDtypeStruct((NW, N), jnp.float32),
        ),
        mesh=mesh,
        scratch_types=[
            pltpu.VMEM((_RPG,), jnp.int32),
            pltpu.VMEM((_RPG, OUT), jnp.float32),
            pltpu.VMEM((_G, OUT), jnp.float32),
            pltpu.VMEM((_G, OUT), jnp.float32),
            pltpu.VMEM((N,), jnp.float32),
            pltpu.SemaphoreType.DMA,
        ],
        compiler_params=pltpu.CompilerParams(needs_layout_passes=False),
    )
    def body(idx_hbm, t_hbm, z_hbm, mmax_hbm, mmin_hbm, cnt_hbm,
             idx_v, rows_v, mx_v, mn_v, cnt_v, sem):
        wid = lax.axis_index("s") * SC_CORES + lax.axis_index("c")
        qbase = wid * q_per
        pltpu.sync_copy(z_hbm, cnt_v)
        ones = jnp.ones((16,), jnp.float32)

        @pl.loop(0, ng)
        def _(g):
            row0 = (qbase + g * _G) * K
            pltpu.sync_copy(idx_hbm.at[pl.ds(row0, _RPG)], idx_v)
            pltpu.async_copy(t_hbm.at[idx_v], rows_v, sem).wait()
            for ql in range(_G):
                ids = idx_v[pl.ds(ql * K, 16)]
                plsc.addupdate_scatter(cnt_v, [ids], ones)
                for dch in range(OUT // 16):
                    sl = pl.ds(dch * 16, 16)
                    a = rows_v[ql * K, sl]
                    mx = a
                    mn = a
                    for r in range(1, K):
                        v = rows_v[ql * K + r, sl]
                        mx = jnp.maximum(mx, v)
                        mn = jnp.minimum(mn, v)
                    mx_v[ql, sl] = mx
                    mn_v[ql, sl] = mn
            pltpu.sync_copy(mx_v, mmax_hbm.at[pl.ds(qbase + g * _G, _G)])
            pltpu.sync_copy(mn_v, mmin_hbm.at[pl.ds(qbase + g * _G, _G)])

        pltpu.sync_copy(cnt_v, cnt_hbm.at[wid])

    return body(idx_flat, T, zeros_i32)


def _stats_body(cnt_ref, t_ref, mean_ref, rstd_ref):
    cf = cnt_ref[...]
    tb = t_ref[...]
    s1p = jax.lax.dot_general(
        cf, tb, (((1,), (0,)), ((), ())), preferred_element_type=jnp.float32)
    s2p = jax.lax.dot_general(
        cf, tb * tb, (((1,), (0,)), ((), ())),
        preferred_element_type=jnp.float32)
    total = jnp.float32(N_NEW * K)
    s1 = jnp.sum(s1p, axis=0, keepdims=True)
    s2 = jnp.sum(s2p, axis=0, keepdims=True)
    mean = s1 / total
    var = s2 / total - mean * mean
    mean_ref[...] = mean
    rstd_ref[...] = lax.rsqrt(var + EPS)


def _stats(cnt_partials, T):
    return pl.pallas_call(
        _stats_body,
        out_shape=[
            jax.ShapeDtypeStruct((1, OUT), jnp.float32),
            jax.ShapeDtypeStruct((1, OUT), jnp.float32),
        ],
    )(cnt_partials, T)


_FT = 512


def _final_body(mx_ref, mn_ref, mean_ref, rstd_ref, g_ref, b_ref, out_ref):
    g = g_ref[...]
    m = jnp.where(g >= 0.0, mx_ref[...], mn_ref[...])
    y = g * ((m - mean_ref[...]) * rstd_ref[...]) + b_ref[...]
    out_ref[...] = jnp.maximum(y, 0.0)


def _finalize(mmax, mmin, mean, rstd, gamma2, beta2):
    grid = (N_NEW // _FT,)
    return pl.pallas_call(
        _final_body,
        grid=grid,
        in_specs=[
            pl.BlockSpec((_FT, OUT), lambda i: (i, 0)),
            pl.BlockSpec((_FT, OUT), lambda i: (i, 0)),
            pl.BlockSpec((1, OUT), lambda i: (0, 0)),
            pl.BlockSpec((1, OUT), lambda i: (0, 0)),
            pl.BlockSpec((1, OUT), lambda i: (0, 0)),
            pl.BlockSpec((1, OUT), lambda i: (0, 0)),
        ],
        out_specs=pl.BlockSpec((_FT, OUT), lambda i: (i, 0)),
        out_shape=jax.ShapeDtypeStruct((N_NEW, OUT), jnp.float32),
    )(mmax, mmin, mean, rstd, gamma2, beta2)


def kernel(p, x, o, W, gamma, beta):
    px = p[:, 0].reshape(128, 128)
    py = p[:, 1].reshape(128, 128)
    pz = p[:, 2].reshape(128, 128)
    fps = _fps(px, py, pz).reshape(N_NEW)

    p_pad128 = jnp.pad(p, ((0, 0), (0, 125)))
    np_pad = _newp_gather(fps, p_pad128)
    new_p = np_pad[:, :3]

    idx = _knn(np_pad[:, :16], jnp.pad(p, ((0, 0), (0, 13))).T)

    xcat = jnp.concatenate([p, x], axis=1)
    T = _linear(xcat, W)

    zeros_f32 = jnp.zeros((N,), jnp.float32)
    mmax, mmin, cntp = _group_stage(idx.reshape(N_NEW * K), T, zeros_f32)

    mean, rstd = _stats(cntp, T)
    x_out = _finalize(mmax, mmin, mean, rstd,
                      gamma.reshape(1, OUT), beta.reshape(1, OUT))
    return (new_p, x_out, o)


# speedup vs baseline: 34.2273x; 1.2203x over previous
import functools

import jax
import jax.numpy as jnp
from jax import lax
from jax.experimental import pallas as pl
from jax.experimental.pallas import tpu as pltpu
from jax.experimental.pallas import tpu_sc as plsc

N = 16384
N_NEW = 4096
K = 16
OUT = 256
FAN_IN = 131
EPS = 1e-5

SC_CORES = 2
SC_SUBCORES = 16
NW = SC_CORES * SC_SUBCORES

BIG_I32 = 2 ** 30


def _fps_body(px_ref, py_ref, pz_ref, out_ref):
    fi_out = lax.broadcasted_iota(jnp.int32, (32, 128), 0) * 128 + \
        lax.broadcasted_iota(jnp.int32, (32, 128), 1)
    fi_p = lax.broadcasted_iota(jnp.int32, (128, 128), 0) * 128 + \
        lax.broadcasted_iota(jnp.int32, (128, 128), 1)
    px = px_ref[...]
    py = py_ref[...]
    pz = pz_ref[...]

    out_ref[...] = jnp.where(fi_out == 0, jnp.int32(0), jnp.int32(-1))
    qx0 = jnp.sum(jnp.where(fi_p == 0, px, 0.0))
    qy0 = jnp.sum(jnp.where(fi_p == 0, py, 0.0))
    qz0 = jnp.sum(jnp.where(fi_p == 0, pz, 0.0))

    def cond(c):
        i, _, _, _, _, _, found = c
        return jnp.logical_and(i < N_NEW, jnp.logical_not(found))

    def body(c):
        i, qx, qy, qz, _, _, _ = c
        d = (px - qx) ** 2 + (py - qy) ** 2 + (pz - qz) ** 2
        m = jnp.max(d)
        nxt = jnp.min(jnp.where(d == m, fi_p, BIG_I32))
        chain = out_ref[...]
        occ = jnp.min(jnp.where(chain == nxt, fi_out, BIG_I32))
        found = occ < BIG_I32
        start = jnp.where(found, occ, 0)
        period = jnp.where(found, i - occ, 0)
        out_ref[...] = jnp.where(
            jnp.logical_and(fi_out == i, jnp.logical_not(found)), nxt, chain)
        nqx = jnp.sum(jnp.where(fi_p == nxt, px, 0.0))
        nqy = jnp.sum(jnp.where(fi_p == nxt, py, 0.0))
        nqz = jnp.sum(jnp.where(fi_p == nxt, pz, 0.0))
        i_next = jnp.where(found, i, i + 1)
        return (i_next, nqx, nqy, nqz, start, period, found)

    init = (jnp.int32(1), qx0, qy0, qz0, jnp.int32(0), jnp.int32(0),
            jnp.bool_(False))
    _, _, _, _, start, period, found = lax.while_loop(cond, body, init)

    def fill(j, _):
        chain = out_ref[...]
        val = jnp.sum(jnp.where(fi_out == start + j, chain, 0))
        mask = jnp.logical_and(fi_out >= start,
                               (fi_out - start) % period == j)
        out_ref[...] = jnp.where(mask, val, chain)
        return 0

    lax.fori_loop(0, period, fill, 0)


def _fps(px, py, pz):
    return pl.pallas_call(
        _fps_body,
        out_shape=jax.ShapeDtypeStruct((32, 128), jnp.int32),
    )(px, py, pz)


def _newp_gather(fps, p_pad):
    b_per = N_NEW // NW
    mesh = plsc.VectorSubcoreMesh(core_axis_name="c", subcore_axis_name="s")

    @functools.partial(
        pl.kernel,
        out_type=jax.ShapeDtypeStruct((N_NEW, 128), jnp.float32),
        mesh=mesh,
        scratch_types=[
            pltpu.VMEM((b_per,), jnp.int32),
            pltpu.VMEM((b_per, 128), jnp.float32),
            pltpu.SemaphoreType.DMA,
        ],
    )
    def body(fps_hbm, ppad_hbm, out_hbm, idx_v, rows_v, sem):
        wid = lax.axis_index("s") * SC_CORES + lax.axis_index("c")
        base = wid * b_per
        pltpu.sync_copy(fps_hbm.at[pl.ds(base, b_per)], idx_v)
        pltpu.async_copy(ppad_hbm.at[idx_v], rows_v, sem).wait()
        pltpu.sync_copy(rows_v, out_hbm.at[pl.ds(base, b_per)])

    return body(fps, p_pad)


_QT = 64


_NC = 128
_CW = N // _NC


def _knn_body(np_ref, pt_ref, out_ref):
    pt = pt_ref[...]
    pn = jnp.sum(pt * pt, axis=0, keepdims=True)
    q = np_ref[...]
    qn = jnp.sum(q * q, axis=1, keepdims=True)
    dot = jax.lax.dot_general(q, pt, (((1,), (0,)), ((), ())),
                              preferred_element_type=jnp.float32)
    d = qn + pn - 2.0 * dot
    inf = jnp.float32(jnp.inf)

    d3 = d.reshape(_QT, _NC, _CW)
    cm = jnp.min(d3, axis=2)
    ci = lax.broadcasted_iota(jnp.int32, (_QT, _NC), 1)
    sel_list = []
    for _ in range(K):
        m = jnp.min(cm, axis=1, keepdims=True)
        sc = jnp.min(jnp.where(cm == m, ci, BIG_I32), axis=1, keepdims=True)
        sel_list.append(sc)
        cm = jnp.where(ci == sc, inf, cm)
    selc = jnp.concatenate(sel_list, axis=1)
    sorted_list = []
    for _ in range(K):
        m = jnp.min(selc, axis=1, keepdims=True)
        sorted_list.append(m)
        selc = jnp.where(selc == m, BIG_I32, selc)
    sels = jnp.concatenate(sorted_list, axis=1)

    c3 = lax.broadcasted_iota(jnp.int32, (_QT, K, _NC), 2)
    oh = (c3 == sels[:, :, None]).astype(jnp.float32)
    dc3 = jax.lax.dot_general(oh, d3, (((2,), (1,)), ((0,), (0,))),
                              preferred_element_type=jnp.float32)
    dc = dc3.reshape(_QT, K * _CW)
    col2 = lax.broadcasted_iota(jnp.int32, (_QT, K * _CW), 1)
    slot_i = lax.broadcasted_iota(jnp.int32, (_QT, K), 1)
    for r in range(K):
        m = jnp.min(dc, axis=1, keepdims=True)
        s2 = jnp.min(jnp.where(dc == m, col2, BIG_I32), axis=1, keepdims=True)
        slot = s2 // _CW
        j = s2 % _CW
        ch = jnp.min(jnp.where(slot_i == slot, sels, BIG_I32), axis=1,
                     keepdims=True)
        out_ref[:, pl.ds(r, 1)] = ch * _CW + j
        dc = jnp.where(col2 == s2, inf, dc)


def _knn(np_pad, p_padT):
    grid = (N_NEW // _QT,)
    return pl.pallas_call(
        _knn_body,
        grid=grid,
        in_specs=[
            pl.BlockSpec((_QT, 16), lambda i: (i, 0)),
            pl.BlockSpec((16, N), lambda i: (0, 0)),
        ],
        out_specs=pl.BlockSpec((_QT, K), lambda i: (i, 0)),
        out_shape=jax.ShapeDtypeStruct((N_NEW, K), jnp.int32),
    )(np_pad, p_padT)


_RT = 2048


def _lin_body(xc_ref, w_ref, out_ref):
    out_ref[...] = jax.lax.dot_general(
        xc_ref[...], w_ref[...], (((1,), (1,)), ((), ())),
        preferred_element_type=jnp.float32)


def _linear(xcat, W):
    grid = (N // _RT,)
    return pl.pallas_call(
        _lin_body,
        grid=grid,
        in_specs=[
            pl.BlockSpec((_RT, FAN_IN), lambda i: (i, 0)),
            pl.BlockSpec((OUT, FAN_IN), lambda i: (0, 0)),
        ],
        out_specs=pl.BlockSpec((_RT, OUT), lambda i: (i, 0)),
        out_shape=jax.ShapeDtypeStruct((N, OUT), jnp.float32),
    )(xcat, W)


_G = 4
_RPG = _G * K


def _group_stage(idx_flat, T, zeros_i32):
    q_per = N_NEW // NW
    ng = q_per // _G
    mesh = plsc.VectorSubcoreMesh(core_axis_name="c", subcore_axis_name="s")

    @functools.partial(
        pl.kernel,
        out_type=(
            jax.ShapeDtypeStruct((N_NEW, OUT), jnp.float32),
            jax.ShapeDtypeStruct((N_NEW, OUT), jnp.float32),
            jax.ShapeDtypeStruct((NW, N), jnp.float32),
        ),
        mesh=mesh,
        scratch_types=[
            pltpu.VMEM((_RPG,), jnp.int32),
            pltpu.VMEM((_RPG, OUT), jnp.float32),
            pltpu.VMEM((_G, OUT), jnp.float32),
            pltpu.VMEM((_G, OUT), jnp.float32),
            pltpu.VMEM((N,), jnp.float32),
            pltpu.SemaphoreType.DMA,
        ],
        compiler_params=pltpu.CompilerParams(needs_layout_passes=False),
    )
    def body(idx_hbm, t_hbm, z_hbm, mmax_hbm, mmin_hbm, cnt_hbm,
             idx_v, rows_v, mx_v, mn_v, cnt_v, sem):
        wid = lax.axis_index("s") * SC_CORES + lax.axis_index("c")
        qbase = wid * q_per
        pltpu.sync_copy(z_hbm, cnt_v)
        ones = jnp.ones((16,), jnp.float32)

        @pl.loop(0, ng)
        def _(g):
            row0 = (qbase + g * _G) * K
            pltpu.sync_copy(idx_hbm.at[pl.ds(row0, _RPG)], idx_v)
            pltpu.async_copy(t_hbm.at[idx_v], rows_v, sem).wait()
            for ql in range(_G):
                ids = idx_v[pl.ds(ql * K, 16)]
                plsc.addupdate_scatter(cnt_v, [ids], ones)
                for dch in range(OUT // 16):
                    sl = pl.ds(dch * 16, 16)
                    a = rows_v[ql * K, sl]
                    mx = a
                    mn = a
                    for r in range(1, K):
                        v = rows_v[ql * K + r, sl]
                        mx = jnp.maximum(mx, v)
                        mn = jnp.minimum(mn, v)
                    mx_v[ql, sl] = mx
                    mn_v[ql, sl] = mn
            pltpu.sync_copy(mx_v, mmax_hbm.at[pl.ds(qbase + g * _G, _G)])
            pltpu.sync_copy(mn_v, mmin_hbm.at[pl.ds(qbase + g * _G, _G)])

        pltpu.sync_copy(cnt_v, cnt_hbm.at[wid])

    return body(idx_flat, T, zeros_i32)


def _stats_body(cnt_ref, t_ref, mean_ref, rstd_ref):
    cf = cnt_ref[...]
    tb = t_ref[...]
    s1p = jax.lax.dot_general(
        cf, tb, (((1,), (0,)), ((), ())), preferred_element_type=jnp.float32)
    s2p = jax.lax.dot_general(
        cf, tb * tb, (((1,), (0,)), ((), ())),
        preferred_element_type=jnp.float32)
    total = jnp.float32(N_NEW * K)
    s1 = jnp.sum(s1p, axis=0, keepdims=True)
    s2 = jnp.sum(s2p, axis=0, keepdims=True)
    mean = s1 / total
    var = s2 / total - mean * mean
    mean_ref[...] = mean
    rstd_ref[...] = lax.rsqrt(var + EPS)


def _stats(cnt_partials, T):
    return pl.pallas_call(
        _stats_body,
        out_shape=[
            jax.ShapeDtypeStruct((1, OUT), jnp.float32),
            jax.ShapeDtypeStruct((1, OUT), jnp.float32),
        ],
    )(cnt_partials, T)


_FT = 512


def _final_body(mx_ref, mn_ref, mean_ref, rstd_ref, g_ref, b_ref, out_ref):
    g = g_ref[...]
    m = jnp.where(g >= 0.0, mx_ref[...], mn_ref[...])
    y = g * ((m - mean_ref[...]) * rstd_ref[...]) + b_ref[...]
    out_ref[...] = jnp.maximum(y, 0.0)


def _finalize(mmax, mmin, mean, rstd, gamma2, beta2):
    grid = (N_NEW // _FT,)
    return pl.pallas_call(
        _final_body,
        grid=grid,
        in_specs=[
            pl.BlockSpec((_FT, OUT), lambda i: (i, 0)),
            pl.BlockSpec((_FT, OUT), lambda i: (i, 0)),
            pl.BlockSpec((1, OUT), lambda i: (0, 0)),
            pl.BlockSpec((1, OUT), lambda i: (0, 0)),
            pl.BlockSpec((1, OUT), lambda i: (0, 0)),
            pl.BlockSpec((1, OUT), lambda i: (0, 0)),
        ],
        out_specs=pl.BlockSpec((_FT, OUT), lambda i: (i, 0)),
        out_shape=jax.ShapeDtypeStruct((N_NEW, OUT), jnp.float32),
    )(mmax, mmin, mean, rstd, gamma2, beta2)


def kernel(p, x, o, W, gamma, beta):
    px = p[:, 0].reshape(128, 128)
    py = p[:, 1].reshape(128, 128)
    pz = p[:, 2].reshape(128, 128)
    fps = _fps(px, py, pz).reshape(N_NEW)

    p_pad128 = jnp.pad(p, ((0, 0), (0, 125)))
    np_pad = _newp_gather(fps, p_pad128)
    new_p = np_pad[:, :3]

    idx = _knn(np_pad[:, :16], jnp.pad(p, ((0, 0), (0, 13))).T)

    xcat = jnp.concatenate([p, x], axis=1)
    T = _linear(xcat, W)

    zeros_f32 = jnp.zeros((N,), jnp.float32)
    mmax, mmin, cntp = _group_stage(idx.reshape(N_NEW * K), T, zeros_f32)

    mean, rstd = _stats(cntp, T)
    x_out = _finalize(mmax, mmin, mean, rstd,
                      gamma.reshape(1, OUT), beta.reshape(1, OUT))
    return (new_p, x_out, o)
